# baseline (speedup 1.0000x reference)
import jax
import jax.numpy as jnp
from jax import lax
from jax.experimental import pallas as pl
from jax.experimental.pallas import tpu as pltpu
from jax.experimental.pallas import tpu_sc as plsc

_B = 16384
_EMB = 64
_NC = 2
_NS = 16
_NW = _NC * _NS
_BPW = _B // _NW
_CHUNK = 128
_NCHUNK = _BPW // _CHUNK
_GROUPS = _BPW // 16


def _mf_body(u_id, i_id, user_emb, user_bias16, item_emb, item_bias16, mean,
             out, uidx_v, iidx_v, ubidx_v, ibidx_v, u_rows, i_rows,
             bu_v, bi_v, out_v, mean_v, sem):
    wid = lax.axis_index("s") * _NC + lax.axis_index("c")
    base = wid * _BPW

    idx_copies = []
    for c in range(_NCHUNK):
        src = pl.ds(base + c * _CHUNK, _CHUNK)
        idx_copies.append(pltpu.async_copy(u_id.at[src], uidx_v.at[c], sem))
        idx_copies.append(pltpu.async_copy(i_id.at[src], iidx_v.at[c], sem))
    idx_copies.append(pltpu.async_copy(mean, mean_v.at[pl.ds(0, 1)], sem))
    for d in idx_copies:
        d.wait()

    gathers = []
    for c in range(_NCHUNK):
        dst = pl.ds(c * _CHUNK, _CHUNK)
        gathers.append(
            pltpu.async_copy(user_emb.at[uidx_v.at[c]], u_rows.at[dst], sem))
        gathers.append(
            pltpu.async_copy(item_emb.at[iidx_v.at[c]], i_rows.at[dst], sem))

    for c in range(_NCHUNK):
        for j in range(_CHUNK // 16):
            s = pl.ds(j * 16, 16)
            ubidx_v[c, s] = lax.shift_right_logical(uidx_v[c, s], 4)
            ibidx_v[c, s] = lax.shift_right_logical(iidx_v[c, s], 4)
    for c in range(_NCHUNK):
        dst = pl.ds(c * _CHUNK, _CHUNK)
        gathers.append(
            pltpu.async_copy(user_bias16.at[ubidx_v.at[c]], bu_v.at[dst], sem))
        gathers.append(
            pltpu.async_copy(item_bias16.at[ibidx_v.at[c]], bi_v.at[dst], sem))
    for d in gathers:
        d.wait()

    mv = mean_v[...]
    mean_vec = jnp.full((16,), mv[0], jnp.float32)
    mask15 = jnp.full((16,), 15, jnp.int32)

    def group_body(g, carry):
        rows = g * 16 + lax.iota(jnp.int32, 16)
        accs = [jnp.zeros((16,), jnp.float32) for _ in range(4)]
        for d in range(_EMB):
            dv = jnp.full((16,), d, jnp.int32)
            u = plsc.load_gather(u_rows, [rows, dv])
            it = plsc.load_gather(i_rows, [rows, dv])
            accs[d % 4] = accs[d % 4] + u * it
        acc = (accs[0] + accs[1]) + (accs[2] + accs[3])
        chunk = lax.shift_right_logical(rows, 7)
        pos = lax.bitwise_and(rows, jnp.full((16,), 127, jnp.int32))
        uids = plsc.load_gather(uidx_v, [chunk, pos])
        iids = plsc.load_gather(iidx_v, [chunk, pos])
        bu = plsc.load_gather(bu_v, [rows, lax.bitwise_and(uids, mask15)])
        bi = plsc.load_gather(bi_v, [rows, lax.bitwise_and(iids, mask15)])
        res = acc + bu + bi + mean_vec
        plsc.store_scatter(out_v, [rows], res)
        return carry

    lax.fori_loop(0, _GROUPS, group_body, 0)

    pltpu.sync_copy(out_v, out.at[pl.ds(base, _BPW)])


@jax.jit
def kernel(u_id, i_id, user_emb, user_bias, item_emb, item_bias, mean):
    mesh = plsc.VectorSubcoreMesh(
        core_axis_name="c", subcore_axis_name="s",
        num_cores=_NC, num_subcores=_NS)
    nu = user_bias.shape[0]
    ni = item_bias.shape[0]
    user_bias16 = user_bias.reshape(nu // 16, 16)
    item_bias16 = item_bias.reshape(ni // 16, 16)
    f = pl.kernel(
        _mf_body,
        out_type=jax.ShapeDtypeStruct((_B,), jnp.float32),
        mesh=mesh,
        compiler_params=pltpu.CompilerParams(
            needs_layout_passes=False, use_tc_tiling_on_sc=False),
        scratch_types=[
            pltpu.VMEM((_NCHUNK, _CHUNK), jnp.int32),
            pltpu.VMEM((_NCHUNK, _CHUNK), jnp.int32),
            pltpu.VMEM((_NCHUNK, _CHUNK), jnp.int32),
            pltpu.VMEM((_NCHUNK, _CHUNK), jnp.int32),
            pltpu.VMEM((_BPW, _EMB), jnp.float32),
            pltpu.VMEM((_BPW, _EMB), jnp.float32),
            pltpu.VMEM((_BPW, 16), jnp.float32),
            pltpu.VMEM((_BPW, 16), jnp.float32),
            pltpu.VMEM((_BPW,), jnp.float32),
            pltpu.VMEM((16,), jnp.float32),
            pltpu.SemaphoreType.DMA,
        ],
    )
    return f(u_id, i_id, user_emb, user_bias16, item_emb, item_bias16, mean)


# speedup vs baseline: 1.0502x; 1.0502x over previous
import jax
import jax.numpy as jnp
from jax import lax
from jax.experimental import pallas as pl
from jax.experimental.pallas import tpu as pltpu
from jax.experimental.pallas import tpu_sc as plsc

_B = 16384
_EMB = 64
_ROW = 128
_NC = 2
_NS = 16
_NW = _NC * _NS
_BPW = _B // _NW
_CHUNK = 128
_NCHUNK = _BPW // _CHUNK
_HALF = _BPW // 2
_NBIAS = 1000448 // 128


def _mf_body(u_id, i_id, user_emb128, user_bias128, item_emb128,
             item_bias128, mean, out, uidx_v, iidx_v, u_rows, i_rows,
             bias_rows, bu_v, bi_v, out_v, mean_v, sem, bsem):
    wid = lax.axis_index("s") * _NC + lax.axis_index("c")
    base = wid * _BPW

    idx_copies = []
    for c in range(_NCHUNK):
        src = pl.ds(base + c * _CHUNK, _CHUNK)
        idx_copies.append(pltpu.async_copy(u_id.at[src], uidx_v.at[c], sem))
        idx_copies.append(pltpu.async_copy(i_id.at[src], iidx_v.at[c], sem))
    idx_copies.append(pltpu.async_copy(mean, mean_v.at[pl.ds(0, 1)], sem))
    for d in idx_copies:
        d.wait()

    for c in range(_NCHUNK):
        for j in range(_CHUNK // 16):
            s = pl.ds(j * 16, 16)
            uidx_v[c + _NCHUNK, s] = lax.shift_right_logical(uidx_v[c, s], 7)
            iidx_v[c + _NCHUNK, s] = lax.shift_right_logical(iidx_v[c, s], 7)

    mv = mean_v[...]
    mean_vec = jnp.full((16,), mv[0], jnp.float32)
    mask127 = jnp.full((16,), 127, jnp.int32)

    def bias_pass(idx_ref, table, dst_ref):
        for c in range(_NCHUNK):
            pltpu.async_copy(table.at[idx_ref.at[c + _NCHUNK]],
                             bias_rows, bsem).wait()
            for j in range(_CHUNK // 16):
                rows = jnp.full((16,), j * 16, jnp.int32) + lax.iota(
                    jnp.int32, 16)
                ids = idx_ref[c, pl.ds(j * 16, 16)]
                lanes = lax.bitwise_and(ids, mask127)
                v = plsc.load_gather(bias_rows, [rows, lanes])
                dst_ref[pl.ds(c * _CHUNK + j * 16, 16)] = v

    bias_pass(uidx_v, user_bias128, bu_v)
    bias_pass(iidx_v, item_bias128, bi_v)

    for h in range(2):
        gathers = []
        for cc in range(_NCHUNK // 2):
            c = h * (_NCHUNK // 2) + cc
            dst = pl.ds(cc * _CHUNK, _CHUNK)
            gathers.append(pltpu.async_copy(
                user_emb128.at[uidx_v.at[c]], u_rows.at[dst], sem))
            gathers.append(pltpu.async_copy(
                item_emb128.at[iidx_v.at[c]], i_rows.at[dst], sem))
        for d in gathers:
            d.wait()

        def group_body(g, carry):
            rows = g * 16 + lax.iota(jnp.int32, 16)
            accs = [jnp.zeros((16,), jnp.float32) for _ in range(4)]
            for d in range(_EMB):
                dv = jnp.full((16,), d, jnp.int32)
                u = plsc.load_gather(u_rows, [rows, dv])
                it = plsc.load_gather(i_rows, [rows, dv])
                accs[d % 4] = accs[d % 4] + u * it
            acc = (accs[0] + accs[1]) + (accs[2] + accs[3])
            off = h * _HALF
            bu = plsc.load_gather(bu_v, [rows + off])
            bi = plsc.load_gather(bi_v, [rows + off])
            res = acc + bu + bi + mean_vec
            plsc.store_scatter(out_v, [rows + off], res)
            return carry

        lax.fori_loop(0, _HALF // 16, group_body, 0)

    pltpu.sync_copy(out_v, out.at[pl.ds(base, _BPW)])


@jax.jit
def kernel(u_id, i_id, user_emb, user_bias, item_emb, item_bias, mean):
    mesh = plsc.VectorSubcoreMesh(
        core_axis_name="c", subcore_axis_name="s",
        num_cores=_NC, num_subcores=_NS)
    user_emb128 = jnp.pad(user_emb, ((0, 0), (0, _ROW - _EMB)))
    item_emb128 = jnp.pad(item_emb, ((0, 0), (0, _ROW - _EMB)))
    nb = _NBIAS * 128
    user_bias128 = jnp.pad(user_bias[:, 0],
                           (0, nb - user_bias.shape[0])).reshape(_NBIAS, 128)
    item_bias128 = jnp.pad(item_bias[:, 0],
                           (0, nb - item_bias.shape[0])).reshape(_NBIAS, 128)
    f = pl.kernel(
        _mf_body,
        out_type=jax.ShapeDtypeStruct((_B,), jnp.float32),
        mesh=mesh,
        compiler_params=pltpu.CompilerParams(needs_layout_passes=False),
        scratch_types=[
            pltpu.VMEM((2 * _NCHUNK, _CHUNK), jnp.int32),
            pltpu.VMEM((2 * _NCHUNK, _CHUNK), jnp.int32),
            pltpu.VMEM((_HALF, _ROW), jnp.float32),
            pltpu.VMEM((_HALF, _ROW), jnp.float32),
            pltpu.VMEM((_CHUNK, _ROW), jnp.float32),
            pltpu.VMEM((_BPW,), jnp.float32),
            pltpu.VMEM((_BPW,), jnp.float32),
            pltpu.VMEM((_BPW,), jnp.float32),
            pltpu.VMEM((16,), jnp.float32),
            pltpu.SemaphoreType.DMA,
            pltpu.SemaphoreType.DMA,
        ],
    )
    return f(u_id, i_id, user_emb128, user_bias128, item_emb128,
             item_bias128, mean)
